# baseline (speedup 1.0000x reference)
import jax
import jax.numpy as jnp
from jax import lax
from jax.experimental import pallas as pl
from jax.experimental.pallas import tpu as pltpu

N = 10000
F_IN = 128
H = 16
BLK = 80
NBLK = N // BLK
CB = 29
CROWS = CB * BLK
S2STEPS = NBLK - CB
CCH = 136
GRID = 2 * NBLK - CB
VLIM = 64 * 1024 * 1024

_NT = (((1,), (1,)), ((), ()))


def _s1t_body(x_ref, w1_ref, s1t_ref):
    s1t_ref[...] = lax.dot_general(
        w1_ref[...], x_ref[...], (((0,), (1,)), ((), ())),
        preferred_element_type=jnp.float32)


def _gcn_body(s1t_ref, adj_ref, b1_ref, w2_ref, b2_ref, w3_ref, b3_ref,
              out_ref, s2_ref, cache_ref, max_ref):
    g = pl.program_id(0)

    @pl.when(g < NBLK)
    def _():
        a = adj_ref[...]
        t = lax.dot_general(a, s1t_ref[...], _NT,
                            preferred_element_type=jnp.float32)
        h = jnp.maximum(t + b1_ref[...], 0.0)
        s2 = jnp.dot(h, w2_ref[...], preferred_element_type=jnp.float32)
        s2_ref[pl.ds(g * BLK, BLK), :] = s2

        @pl.when(g < CB)
        def _():
            cache_ref[pl.ds(g, 1), :, :] = a.astype(jnp.bfloat16).reshape(
                1, BLK, N)

    @pl.when(g >= NBLK)
    def _():
        u = jnp.dot(adj_ref[...], s2_ref[...],
                    preferred_element_type=jnp.float32)
        m = jnp.max(u, axis=0, keepdims=True)

        prev = jnp.where(g == NBLK, jnp.full((1, 2), -3.0e38, jnp.float32),
                         max_ref[...])
        max_ref[...] = jnp.maximum(prev, m)

        c = g - NBLK

        @pl.when(c < CB)
        def _():
            s2b = s2_ref[...].astype(jnp.bfloat16)
            chunk = cache_ref[pl.ds(c, 1), :, :].reshape(BLK, N)
            uc = jnp.dot(chunk, s2b, preferred_element_type=jnp.float32)
            mc = jnp.max(uc, axis=0, keepdims=True)
            max_ref[...] = jnp.maximum(max_ref[...], mc)

    @pl.when(g == GRID - 1)
    def _():
        pooled = max_ref[...] + b2_ref[...]
        val = (pooled[0, 0] * w3_ref[0, 0] + pooled[0, 1] * w3_ref[1, 0]
               + b3_ref[0, 0])
        out_ref[...] = jnp.full((1, 1, 1), val, jnp.float32)


def kernel(x, adj, W1, b1, W2, b2, W3, b3):
    s1t = pl.pallas_call(
        _s1t_body,
        in_specs=[
            pl.BlockSpec((N, F_IN), lambda: (0, 0)),
            pl.BlockSpec((F_IN, H), lambda: (0, 0)),
        ],
        out_specs=pl.BlockSpec((H, N), lambda: (0, 0)),
        out_shape=jax.ShapeDtypeStruct((H, N), jnp.float32),
    )(x, W1)

    def adj_map(g):
        return (jnp.where(g < NBLK, g, g - (NBLK - CB)), 0)

    return pl.pallas_call(
        _gcn_body,
        grid=(GRID,),
        in_specs=[
            pl.BlockSpec((H, N), lambda g: (0, 0)),
            pl.BlockSpec((BLK, N), adj_map),
            pl.BlockSpec((1, H), lambda g: (0, 0)),
            pl.BlockSpec((H, 2), lambda g: (0, 0)),
            pl.BlockSpec((1, 2), lambda g: (0, 0)),
            pl.BlockSpec((2, 1), lambda g: (0, 0)),
            pl.BlockSpec((1, 1), lambda g: (0, 0)),
        ],
        out_specs=pl.BlockSpec((1, 1, 1), lambda g: (0, 0, 0)),
        out_shape=jax.ShapeDtypeStruct((1, 1, 1), jnp.float32),
        scratch_shapes=[
            pltpu.VMEM((N, 2), jnp.float32),
            pltpu.VMEM((CB, BLK, N), jnp.bfloat16),
            pltpu.VMEM((1, 2), jnp.float32),
        ],
        compiler_params=pltpu.CompilerParams(vmem_limit_bytes=VLIM),
    )(s1t, adj, b1.reshape(1, H), W2, b2.reshape(1, 2), W3,
      b3.reshape(1, 1))


# speedup vs baseline: 1.3058x; 1.3058x over previous
import jax
import jax.numpy as jnp
from jax import lax
from jax.experimental import pallas as pl
from jax.experimental.pallas import tpu as pltpu

N = 10000
F_IN = 128
H = 16
BLK = 200
NBLK = N // BLK
CB = 9
SP = max(1, (N // 200 - CB) // CB)
CROWS = CB * BLK
S2STEPS = NBLK - CB
CCH = 136
GRID = 2 * NBLK - CB
VLIM = 64 * 1024 * 1024

_NT = (((1,), (1,)), ((), ()))


def _s1t_body(x_ref, w1_ref, s1t_ref):
    s1t_ref[...] = lax.dot_general(
        w1_ref[...], x_ref[...], (((0,), (1,)), ((), ())),
        preferred_element_type=jnp.float32)


def _gcn_body(s1t_ref, adj_ref, b1_ref, w2_ref, b2_ref, w3_ref, b3_ref,
              out_ref, s2_ref, s2bt_ref, cache_ref, max_ref):
    g = pl.program_id(0)

    @pl.when(g < NBLK)
    def _():
        a = adj_ref[...]
        t = lax.dot_general(a, s1t_ref[...], _NT,
                            preferred_element_type=jnp.float32)
        h = jnp.maximum(t + b1_ref[...], 0.0)
        s2 = jnp.dot(h, w2_ref[...], preferred_element_type=jnp.float32)
        s2_ref[pl.ds(g * BLK, BLK), :] = s2

        @pl.when(g < CB)
        def _():
            cache_ref[pl.ds(g, 1), :, :] = a.astype(jnp.bfloat16).reshape(
                1, BLK, N)

    @pl.when(g == NBLK)
    def _():
        s2bt_ref[...] = s2_ref[...].T.astype(jnp.bfloat16)

    @pl.when(g >= NBLK)
    def _():
        u = jnp.dot(adj_ref[...], s2_ref[...],
                    preferred_element_type=jnp.float32)
        m = jnp.max(u, axis=0, keepdims=True)

        prev = jnp.where(g == NBLK, jnp.full((1, 2), -3.0e38, jnp.float32),
                         max_ref[...])
        max_ref[...] = jnp.maximum(prev, m)

        c = g - NBLK

        @pl.when((c % SP == 0) & (c < SP * CB))
        def _():
            chunk = cache_ref[pl.ds(c // SP, 1), :, :].reshape(BLK, N)
            uc = lax.dot_general(chunk, s2bt_ref[...], _NT,
                                 preferred_element_type=jnp.float32)
            mc = jnp.max(uc, axis=0, keepdims=True)
            max_ref[...] = jnp.maximum(max_ref[...], mc)

    @pl.when(g == GRID - 1)
    def _():
        pooled = max_ref[...] + b2_ref[...]
        val = (pooled[0, 0] * w3_ref[0, 0] + pooled[0, 1] * w3_ref[1, 0]
               + b3_ref[0, 0])
        out_ref[...] = jnp.full((1, 1, 1), val, jnp.float32)


def kernel(x, adj, W1, b1, W2, b2, W3, b3):
    s1t = pl.pallas_call(
        _s1t_body,
        in_specs=[
            pl.BlockSpec((N, F_IN), lambda: (0, 0)),
            pl.BlockSpec((F_IN, H), lambda: (0, 0)),
        ],
        out_specs=pl.BlockSpec((H, N), lambda: (0, 0)),
        out_shape=jax.ShapeDtypeStruct((H, N), jnp.float32),
    )(x, W1)

    def adj_map(g):
        return (jnp.where(g < NBLK, g, g - (NBLK - CB)), 0)

    return pl.pallas_call(
        _gcn_body,
        grid=(GRID,),
        in_specs=[
            pl.BlockSpec((H, N), lambda g: (0, 0)),
            pl.BlockSpec((BLK, N), adj_map),
            pl.BlockSpec((1, H), lambda g: (0, 0)),
            pl.BlockSpec((H, 2), lambda g: (0, 0)),
            pl.BlockSpec((1, 2), lambda g: (0, 0)),
            pl.BlockSpec((2, 1), lambda g: (0, 0)),
            pl.BlockSpec((1, 1), lambda g: (0, 0)),
        ],
        out_specs=pl.BlockSpec((1, 1, 1), lambda g: (0, 0, 0)),
        out_shape=jax.ShapeDtypeStruct((1, 1, 1), jnp.float32),
        scratch_shapes=[
            pltpu.VMEM((N, 2), jnp.float32),
            pltpu.VMEM((2, N), jnp.bfloat16),
            pltpu.VMEM((CB, BLK, N), jnp.bfloat16),
            pltpu.VMEM((1, 2), jnp.float32),
        ],
        compiler_params=pltpu.CompilerParams(
            vmem_limit_bytes=VLIM, internal_scratch_in_bytes=1 << 20),
    )(s1t, adj, b1.reshape(1, H), W2, b2.reshape(1, 2), W3,
      b3.reshape(1, 1))
